# baseline (speedup 1.0000x reference)
import jax
import jax.numpy as jnp
from jax import lax
from jax.experimental import pallas as pl
from jax.experimental.pallas import tpu as pltpu
from jax.experimental.pallas import tpu_sc as plsc

B = 16384
N = 20
EMB = 32
NC = 2
NS = 16
NW = NC * NS
BPW = B // NW
C = 32
NCH = BPW // C
CI = C * N
NG = CI // 128
ROWS_PER_W = (BPW * N) // 128
KBLK = 123
SROWS = KBLK * 2048


def _sc_pred_body(users_hbm, bundles_hbm, remb_t, staged_d, out_hbm,
                  uidx, bidx, gidx, ebase, ulane, urows_t, drows, pred,
                  pscr, sem):
    cid = lax.axis_index("c")
    sid = lax.axis_index("s")
    wid = sid * NC + cid
    ubase_w = pl.multiple_of(wid * BPW, BPW)

    pltpu.sync_copy(users_hbm.at[pl.ds(ubase_w, BPW)], uidx)
    pltpu.sync_copy(bundles_hbm.at[pl.ds(wid * ROWS_PER_W, ROWS_PER_W)], bidx)

    iota16 = lax.iota(jnp.int32, 16)
    tr_idx = iota16 * 16
    sub_a = jnp.right_shift(iota16, 3)
    sub_b = jnp.bitwise_and(iota16, 7)

    def block_split(rv):
        m = (jnp.left_shift(jnp.right_shift(rv, 13), 11)
             + jnp.bitwise_and(rv, 2047))
        cb = jnp.left_shift(jnp.bitwise_and(jnp.right_shift(rv, 11), 3), 5)
        return m, cb

    def chunk_body(ch, carry):
        u_off = pl.multiple_of(ch * C, C)
        for v in range(CI // 16):
            bv = bidx[ch * NG + v // 8, pl.ds((v % 8) * 16, 16)]
            mv, cv = block_split(bv)
            gidx[pl.ds(v * 16, 16)] = mv
            ebase[pl.ds(v * 16, 16)] = cv
        uvs = [uidx[pl.ds(u_off + v * 16, 16)] for v in range(C // 16)]
        for v in range(C // 16):
            ulane[pl.ds(v * 16, 16)] = jnp.bitwise_and(uvs[v], 15)

        cps = []
        for g in range(NG):
            cps.append(pltpu.async_copy(
                staged_d.at[gidx.at[pl.ds(g * 128, 128)]],
                drows.at[pl.ds(g * 128, 128)], sem))
        for ul in range(C):
            u = uvs[ul // 16][ul % 16]
            col16 = pl.multiple_of(
                jnp.left_shift(jnp.right_shift(u, 4), 4), 16)
            for tr in range(4):
                cps.append(pltpu.async_copy(
                    remb_t.at[pl.ds(tr * 8, 8), pl.ds(col16, 16)],
                    urows_t.at[ul * 4 + tr], sem))
        for cp in cps:
            cp.wait()

        def blk_body(b, carry2):
            ulv = ulane[pl.ds(b * 4, 16)]
            ebv = [ebase[pl.ds(b * 80 + k * 16, 16)] for k in range(5)]
            for uu in range(4):
                ul = b * 4 + uu
                lane = jnp.full((16,), ulv[uu])
                ua = jnp.full((16,), ul * 4) + sub_a
                u0 = plsc.load_gather(urows_t, [ua, sub_b, lane])
                u1 = plsc.load_gather(urows_t, [ua + 2, sub_b, lane])
                for i in range(N):
                    t = uu * N + i
                    drow = jnp.full((16,), b * 80 + t, jnp.int32)
                    db = jnp.full((16,), ebv[t // 16][t % 16]) + iota16
                    d0 = plsc.load_gather(drows, [drow, db])
                    d1 = plsc.load_gather(drows, [drow, db + 16])
                    pscr[pl.ds(t * 16, 16)] = d0 * u0 + d1 * u1
            for k in range(5):
                acc = plsc.load_gather(pscr, [tr_idx + (k * 256)])
                for d in range(1, 16):
                    acc = acc + plsc.load_gather(pscr, [tr_idx + (k * 256 + d)])
                pred[pl.ds(b * 80 + k * 16, 16)] = acc
            return carry2

        lax.fori_loop(0, C // 4, blk_body, 0)
        pltpu.sync_copy(pred, out_hbm.at[pl.ds((ubase_w + u_off) * N, CI)])
        return carry

    lax.fori_loop(0, NCH, chunk_body, 0)


_sc_pred = pl.kernel(
    _sc_pred_body,
    out_type=jax.ShapeDtypeStruct((B * N,), jnp.float32),
    mesh=plsc.VectorSubcoreMesh(core_axis_name="c", subcore_axis_name="s"),
    compiler_params=pltpu.CompilerParams(
        needs_layout_passes=False, use_tc_tiling_on_sc=False),
    scratch_types=[
        pltpu.VMEM((BPW,), jnp.int32),
        pltpu.VMEM((ROWS_PER_W, 128), jnp.int32),
        pltpu.VMEM((CI,), jnp.int32),
        pltpu.VMEM((CI + 16,), jnp.int32),
        pltpu.VMEM((C + 16,), jnp.int32),
        pltpu.VMEM((C * 4, 8, 16), jnp.float32),
        pltpu.VMEM((CI, 128), jnp.float32),
        pltpu.VMEM((CI,), jnp.float32),
        pltpu.VMEM((80 * 16,), jnp.float32),
        pltpu.SemaphoreType.DMA,
    ],
)


def _stage_body(x0, x1, x2, x3, out_ref):
    xh = jnp.concatenate([x0[...], x1[...], x2[...], x3[...]], axis=0)
    r = lax.broadcasted_iota(jnp.int32, (128, 128), 0)
    c = lax.broadcasted_iota(jnp.int32, (128, 128), 1)
    eye = (r == c).astype(jnp.float32)
    dn = (((0,), (0,)), ((), ()))
    out_ref[...] = lax.dot_general(xh, eye, dn,
                                   preferred_element_type=jnp.float32)


_tc_stage = pl.pallas_call(
    _stage_body,
    grid=(KBLK,),
    in_specs=[pl.BlockSpec((EMB, 2048),
                           lambda k, q=q: (0, jnp.minimum(4 * k + q, 488)))
              for q in range(4)],
    out_specs=pl.BlockSpec((2048, 128), lambda k: (k, 0)),
    out_shape=jax.ShapeDtypeStruct((SROWS, 128), jnp.float32),
)


def _loss_body(pred_ref, w_ref, out_ref):
    pred = pred_ref[...]
    w = w_ref[...]
    pos = pred[:, 0:1]
    negs = pred[:, 1:]
    loss = -jax.nn.log_sigmoid(pos - negs) * w
    out_ref[0, 0] = jnp.sum(loss) / (B * (N - 1))


_tc_loss = pl.pallas_call(
    _loss_body,
    out_shape=jax.ShapeDtypeStruct((1, 1), jnp.float32),
    out_specs=pl.BlockSpec(memory_space=pltpu.SMEM),
)


@jax.jit
def kernel(users, bundles, weights, reviewer_emb, diner_emb):
    users_flat = users.reshape(B).astype(jnp.int32)
    bundles2d = bundles.reshape((B * N) // 128, 128).astype(jnp.int32)
    demb_t = diner_emb.T
    staged_d = _tc_stage(demb_t, demb_t, demb_t, demb_t)
    pred = _sc_pred(users_flat, bundles2d, reviewer_emb.T, staged_d)
    loss = _tc_loss(pred.reshape(B, N), weights)
    return loss[0, 0]


# speedup vs baseline: 5.7422x; 5.7422x over previous
import jax
import jax.numpy as jnp
from jax import lax
from jax.experimental import pallas as pl
from jax.experimental.pallas import tpu as pltpu
from jax.experimental.pallas import tpu_sc as plsc

B = 16384
N = 20
EMB = 32
NC = 2
NS = 16
NW = NC * NS
BPW = B // NW
C = 32
NCH = BPW // C
CI = C * N
NG = CI // 128
ROWS_PER_W = (BPW * N) // 128
KBLK = 123
SROWS = KBLK * 2048


def _sc_pred_body(users_hbm, bundles_hbm, staged_r, staged_d, out_hbm,
                  uidx, bidx, gidx, ebase, umidx, ucb, urows, drows, pred,
                  pscr, sem):
    cid = lax.axis_index("c")
    sid = lax.axis_index("s")
    wid = sid * NC + cid
    ubase_w = pl.multiple_of(wid * BPW, BPW)

    pltpu.sync_copy(users_hbm.at[pl.ds(ubase_w, BPW)], uidx)
    pltpu.sync_copy(bundles_hbm.at[pl.ds(wid * ROWS_PER_W, ROWS_PER_W)], bidx)

    iota16 = lax.iota(jnp.int32, 16)
    tr_idx = iota16 * 16

    def block_split(rv):
        m = (jnp.left_shift(jnp.right_shift(rv, 13), 11)
             + jnp.bitwise_and(rv, 2047))
        cb = jnp.left_shift(jnp.bitwise_and(jnp.right_shift(rv, 11), 3), 5)
        return m, cb

    def chunk_body(ch, carry):
        u_off = pl.multiple_of(ch * C, C)
        for v in range(CI // 16):
            bv = bidx[ch * NG + v // 8, pl.ds((v % 8) * 16, 16)]
            mv, cv = block_split(bv)
            gidx[pl.ds(v * 16, 16)] = mv
            ebase[pl.ds(v * 16, 16)] = cv
        for v in range(C // 16):
            mv, cv = block_split(uidx[pl.ds(u_off + v * 16, 16)])
            umidx[pl.ds(v * 16, 16)] = mv
            ucb[pl.ds(v * 16, 16)] = cv

        cps = [pltpu.async_copy(staged_r.at[umidx], urows, sem)]
        for g in range(NG):
            cps.append(pltpu.async_copy(
                staged_d.at[gidx.at[pl.ds(g * 128, 128)]],
                drows.at[pl.ds(g * 128, 128)], sem))
        for cp in cps:
            cp.wait()

        def blk_body(b, carry2):
            ucv = ucb[pl.ds(b * 4, 16)]
            ebv = [ebase[pl.ds(b * 80 + k * 16, 16)] for k in range(5)]
            for uu in range(4):
                urow = jnp.full((16,), b * 4 + uu, jnp.int32)
                ub = jnp.full((16,), ucv[uu]) + iota16
                u0 = plsc.load_gather(urows, [urow, ub])
                u1 = plsc.load_gather(urows, [urow, ub + 16])
                for i in range(N):
                    t = uu * N + i
                    drow = jnp.full((16,), b * 80 + t, jnp.int32)
                    db = jnp.full((16,), ebv[t // 16][t % 16]) + iota16
                    d0 = plsc.load_gather(drows, [drow, db])
                    d1 = plsc.load_gather(drows, [drow, db + 16])
                    pscr[pl.ds(t * 16, 16)] = d0 * u0 + d1 * u1
            for k in range(5):
                acc = plsc.load_gather(pscr, [tr_idx + (k * 256)])
                for d in range(1, 16):
                    acc = acc + plsc.load_gather(pscr, [tr_idx + (k * 256 + d)])
                pred[pl.ds(b * 80 + k * 16, 16)] = acc
            return carry2

        lax.fori_loop(0, C // 4, blk_body, 0)
        pltpu.sync_copy(pred, out_hbm.at[pl.ds((ubase_w + u_off) * N, CI)])
        return carry

    lax.fori_loop(0, NCH, chunk_body, 0)


_sc_pred = pl.kernel(
    _sc_pred_body,
    out_type=jax.ShapeDtypeStruct((B * N,), jnp.float32),
    mesh=plsc.VectorSubcoreMesh(core_axis_name="c", subcore_axis_name="s"),
    compiler_params=pltpu.CompilerParams(
        needs_layout_passes=False, use_tc_tiling_on_sc=False),
    scratch_types=[
        pltpu.VMEM((BPW,), jnp.int32),
        pltpu.VMEM((ROWS_PER_W, 128), jnp.int32),
        pltpu.VMEM((CI,), jnp.int32),
        pltpu.VMEM((CI + 16,), jnp.int32),
        pltpu.VMEM((C,), jnp.int32),
        pltpu.VMEM((C + 16,), jnp.int32),
        pltpu.VMEM((C, 128), jnp.float32),
        pltpu.VMEM((CI, 128), jnp.float32),
        pltpu.VMEM((CI,), jnp.float32),
        pltpu.VMEM((80 * 16,), jnp.float32),
        pltpu.SemaphoreType.DMA,
    ],
)


def _stage_body(x0, x1, x2, x3, out_ref):
    xh = jnp.concatenate([x0[...], x1[...], x2[...], x3[...]], axis=0)
    r = lax.broadcasted_iota(jnp.int32, (128, 128), 0)
    c = lax.broadcasted_iota(jnp.int32, (128, 128), 1)
    eye = (r == c).astype(jnp.float32)
    dn = (((0,), (0,)), ((), ()))
    out_ref[...] = lax.dot_general(xh, eye, dn,
                                   preferred_element_type=jnp.float32)


_tc_stage = pl.pallas_call(
    _stage_body,
    grid=(KBLK,),
    in_specs=[pl.BlockSpec((EMB, 2048),
                           lambda k, q=q: (0, jnp.minimum(4 * k + q, 488)))
              for q in range(4)],
    out_specs=pl.BlockSpec((2048, 128), lambda k: (k, 0)),
    out_shape=jax.ShapeDtypeStruct((SROWS, 128), jnp.float32),
)


def _loss_body(pred_ref, w_ref, out_ref):
    pred = pred_ref[...]
    w = w_ref[...]
    pos = pred[:, 0:1]
    negs = pred[:, 1:]
    loss = -jax.nn.log_sigmoid(pos - negs) * w
    out_ref[0, 0] = jnp.sum(loss) / (B * (N - 1))


_tc_loss = pl.pallas_call(
    _loss_body,
    out_shape=jax.ShapeDtypeStruct((1, 1), jnp.float32),
    out_specs=pl.BlockSpec(memory_space=pltpu.SMEM),
)


@jax.jit
def kernel(users, bundles, weights, reviewer_emb, diner_emb):
    users_flat = users.reshape(B).astype(jnp.int32)
    bundles2d = bundles.reshape((B * N) // 128, 128).astype(jnp.int32)
    demb_t = diner_emb.T
    remb_t = reviewer_emb.T
    staged_d = _tc_stage(demb_t, demb_t, demb_t, demb_t)
    staged_r = _tc_stage(remb_t, remb_t, remb_t, remb_t)
    pred = _sc_pred(users_flat, bundles2d, staged_r, staged_d)
    loss = _tc_loss(pred.reshape(B, N), weights)
    return loss[0, 0]
